# baseline (speedup 1.0000x reference)
import functools

import jax
import jax.numpy as jnp
from jax import lax
from jax.experimental import pallas as pl
from jax.experimental.pallas import tpu as pltpu
from jax.experimental.pallas import tpu_sc as plsc

EMB = 32
BATCH = 16384
NROWS = 1000000
RSTRIDE = 1000064
RMAIN = 999936
NTAIL = NROWS - RMAIN
AUXOFF = EMB * RSTRIDE
FLATN = AUXOFF + EMB * NTAIL

NC = 2
NS = 16
L = 16
NW = NC * NS
BPW = BATCH // NW
KV = BPW // L


def _flatten_body(u_ref, i_ref, ua_ref, ia_ref, uo_ref, io_ref,
                  sem_u, sem_i):
    copies = []
    for src, aux, dst, sem in ((u_ref, ua_ref, uo_ref, sem_u),
                               (i_ref, ia_ref, io_ref, sem_i)):
        for e in range(EMB):
            copies.append(pltpu.make_async_copy(
                src.at[e, pl.ds(0, RMAIN)],
                dst.at[pl.ds(e * RSTRIDE, RMAIN)], sem))
        copies.append(pltpu.make_async_copy(
            aux, dst.at[pl.ds(AUXOFF, EMB * NTAIL)], sem))
    for c in copies:
        c.start()
    for c in copies:
        c.wait()


def _flatten(ut, it):
    ua = ut[:, RMAIN:].reshape(-1)
    ia = it[:, RMAIN:].reshape(-1)
    return pl.pallas_call(
        _flatten_body,
        in_specs=[pl.BlockSpec(memory_space=pl.ANY)] * 4,
        out_specs=[pl.BlockSpec(memory_space=pl.ANY),
                   pl.BlockSpec(memory_space=pl.ANY)],
        out_shape=[jax.ShapeDtypeStruct((FLATN,), jnp.float32),
                   jax.ShapeDtypeStruct((FLATN,), jnp.float32)],
        scratch_shapes=[pltpu.SemaphoreType.DMA, pltpu.SemaphoreType.DMA],
    )(ut, it, ua, ia)


def _body(user_hbm, item_hbm, uid_hbm, iid_hbm, out_hbm, *scr):
    uidx = scr[0:EMB]
    iidx = scr[EMB:2 * EMB]
    ubuf = scr[2 * EMB:3 * EMB]
    ibuf = scr[3 * EMB:4 * EMB]
    outv_v = scr[4 * EMB]
    sem_u = scr[4 * EMB + 1]
    sem_i = scr[4 * EMB + 2]

    wid = lax.axis_index("s") * NC + lax.axis_index("c")
    base = wid * BPW

    pltpu.sync_copy(uid_hbm.at[pl.ds(base, BPW)], uidx[0])
    pltpu.sync_copy(iid_hbm.at[pl.ds(base, BPW)], iidx[0])

    def mkidx(k, carry):
        s = pl.ds(k * L, L)
        u0 = uidx[0][s]
        i0 = iidx[0][s]
        u_tail = u0 >= RMAIN
        i_tail = i0 >= RMAIN
        for e in range(EMB):
            ue = jnp.where(u_tail, AUXOFF + e * NTAIL + (u0 - RMAIN),
                           u0 + e * RSTRIDE)
            ie = jnp.where(i_tail, AUXOFF + e * NTAIL + (i0 - RMAIN),
                           i0 + e * RSTRIDE)
            if e == 0:
                ue0, ie0 = ue, ie
            else:
                uidx[e][s] = ue
                iidx[e][s] = ie
        uidx[0][s] = ue0
        iidx[0][s] = ie0
        return carry

    lax.fori_loop(0, KV, mkidx, 0)

    copies = []
    for e in range(EMB):
        copies.append(pltpu.async_copy(
            user_hbm.at[uidx[e]], ubuf[e], sem_u))
        copies.append(pltpu.async_copy(
            item_hbm.at[iidx[e]], ibuf[e], sem_i))
    for c in copies:
        c.wait()

    def chunk(k, carry):
        s = pl.ds(k * L, L)
        acc = ubuf[0][s] * ibuf[0][s]
        for e in range(1, EMB):
            acc = acc + ubuf[e][s] * ibuf[e][s]
        outv_v[s] = acc
        return carry

    lax.fori_loop(0, KV, chunk, 0)
    pltpu.sync_copy(outv_v, out_hbm.at[pl.ds(base, BPW)])


@jax.jit
def kernel(user_table, item_table, u_id, i_id):
    uflat, iflat = _flatten(user_table.T, item_table.T)
    mesh = plsc.VectorSubcoreMesh(core_axis_name="c", subcore_axis_name="s",
                                  num_cores=NC, num_subcores=NS)
    k = functools.partial(
        pl.kernel,
        out_type=jax.ShapeDtypeStruct((BATCH,), jnp.float32),
        mesh=mesh,
        scratch_types=(
            [pltpu.VMEM((BPW,), jnp.int32) for _ in range(2 * EMB)]
            + [pltpu.VMEM((BPW,), jnp.float32) for _ in range(2 * EMB)]
            + [pltpu.VMEM((BPW,), jnp.float32),
               pltpu.SemaphoreType.DMA,
               pltpu.SemaphoreType.DMA]
        ),
    )(_body)
    return k(uflat, iflat, u_id.astype(jnp.int32), i_id.astype(jnp.int32))


# speedup vs baseline: 1.5220x; 1.5220x over previous
import functools

import jax
import jax.numpy as jnp
from jax import lax
from jax.experimental import pallas as pl
from jax.experimental.pallas import tpu as pltpu
from jax.experimental.pallas import tpu_sc as plsc

EMB = 32
BATCH = 16384
NROWS = 1000000

NC = 2
NS = 16
L = 16
NW = NC * NS
BPW = BATCH // NW
KV = BPW // L


def _body(user_hbm, item_hbm, uid_hbm, iid_hbm, out_hbm, *scr):
    uidx_v = scr[0]
    iidx_v = scr[1]
    ubuf = scr[2:2 + EMB]
    ibuf = scr[2 + EMB:2 + 2 * EMB]
    outv_v = scr[2 + 2 * EMB]
    sem_u = scr[3 + 2 * EMB]
    sem_i = scr[4 + 2 * EMB]

    wid = lax.axis_index("s") * NC + lax.axis_index("c")
    base = wid * BPW

    pltpu.sync_copy(uid_hbm.at[pl.ds(base, BPW)], uidx_v)
    pltpu.sync_copy(iid_hbm.at[pl.ds(base, BPW)], iidx_v)

    copies = []
    for e in range(EMB):
        copies.append(pltpu.async_copy(
            user_hbm.at[e].at[uidx_v], ubuf[e], sem_u))
        copies.append(pltpu.async_copy(
            item_hbm.at[e].at[iidx_v], ibuf[e], sem_i))
    for c in copies:
        c.wait()

    def chunk(k, carry):
        s = pl.ds(k * L, L)
        acc = ubuf[0][s] * ibuf[0][s]
        for e in range(1, EMB):
            acc = acc + ubuf[e][s] * ibuf[e][s]
        outv_v[s] = acc
        return carry

    lax.fori_loop(0, KV, chunk, 0)
    pltpu.sync_copy(outv_v, out_hbm.at[pl.ds(base, BPW)])


@jax.jit
def kernel(user_table, item_table, u_id, i_id):
    mesh = plsc.VectorSubcoreMesh(core_axis_name="c", subcore_axis_name="s",
                                  num_cores=NC, num_subcores=NS)
    k = functools.partial(
        pl.kernel,
        out_type=jax.ShapeDtypeStruct((BATCH,), jnp.float32),
        mesh=mesh,
        scratch_types=(
            [pltpu.VMEM((BPW,), jnp.int32) for _ in range(2)]
            + [pltpu.VMEM((BPW,), jnp.float32) for _ in range(2 * EMB)]
            + [pltpu.VMEM((BPW,), jnp.float32),
               pltpu.SemaphoreType.DMA,
               pltpu.SemaphoreType.DMA]
        ),
        compiler_params=pltpu.CompilerParams(use_tc_tiling_on_sc=False),
    )(_body)
    return k(user_table.T, item_table.T,
             u_id.astype(jnp.int32), i_id.astype(jnp.int32))


# speedup vs baseline: 29.8192x; 19.5922x over previous
import functools

import jax
import jax.numpy as jnp
from jax import lax
from jax.experimental import pallas as pl
from jax.experimental.pallas import tpu as pltpu
from jax.experimental.pallas import tpu_sc as plsc

EMB = 32
BATCH = 16384
NROWS = 1000000
RSTRIDE = 1000064
RMAIN = 999936
NTAIL = NROWS - RMAIN
AUXOFF = EMB * RSTRIDE
FLATN = AUXOFF + EMB * NTAIL

NC = 2
NS = 16
L = 16
NW = NC * NS
BPW = BATCH // NW
KV = BPW // L

CW = RMAIN // 4
SW = 8064
NSLAB = CW // SW


def _detile_unit(src, dst, i, c, buf, wsem):
    def step(j, carry):
        col = pl.multiple_of(c * CW + j * SW, 128)
        pltpu.sync_copy(src.at[pl.ds(8 * i, 8), pl.ds(col, SW)], buf)
        ws = []
        for s in range(8):
            off = pl.multiple_of((8 * i + s) * RSTRIDE + col, 128)
            ws.append(pltpu.async_copy(
                buf.at[s], dst.at[pl.ds(off, SW)], wsem))
        for w in ws:
            w.wait()
        return carry

    lax.fori_loop(0, NSLAB, step, 0)


def _detile_body(ut_hbm, it_hbm, ua_hbm, ia_hbm, uo_hbm, io_hbm,
                 buf, wsem, asem):
    wid = lax.axis_index("s") * NC + lax.axis_index("c")
    t = wid // 16
    u = wid % 16
    i = u // 4
    c = u % 4

    @pl.when(t == 0)
    def _():
        _detile_unit(ut_hbm, uo_hbm, i, c, buf, wsem)

    @pl.when(t == 1)
    def _():
        _detile_unit(it_hbm, io_hbm, i, c, buf, wsem)

    @pl.when(wid == 0)
    def _():
        pltpu.async_copy(ua_hbm, uo_hbm.at[pl.ds(AUXOFF, EMB * NTAIL)],
                         asem).wait()

    @pl.when(wid == 1)
    def _():
        pltpu.async_copy(ia_hbm, io_hbm.at[pl.ds(AUXOFF, EMB * NTAIL)],
                         asem).wait()


def _gather_body(user_hbm, item_hbm, uid_hbm, iid_hbm, out_hbm, *scr):
    uidx = scr[0:EMB]
    iidx = scr[EMB:2 * EMB]
    ubuf = scr[2 * EMB:3 * EMB]
    ibuf = scr[3 * EMB:4 * EMB]
    outv_v = scr[4 * EMB]
    sem_u = scr[4 * EMB + 1]
    sem_i = scr[4 * EMB + 2]

    wid = lax.axis_index("s") * NC + lax.axis_index("c")
    base = wid * BPW

    pltpu.sync_copy(uid_hbm.at[pl.ds(base, BPW)], uidx[0])
    pltpu.sync_copy(iid_hbm.at[pl.ds(base, BPW)], iidx[0])

    def mkidx(k, carry):
        s = pl.ds(k * L, L)
        u0 = uidx[0][s]
        i0 = iidx[0][s]
        u_tail = u0 >= RMAIN
        i_tail = i0 >= RMAIN
        for e in range(EMB):
            ue = jnp.where(u_tail, AUXOFF + e * NTAIL + (u0 - RMAIN),
                           u0 + e * RSTRIDE)
            ie = jnp.where(i_tail, AUXOFF + e * NTAIL + (i0 - RMAIN),
                           i0 + e * RSTRIDE)
            if e == 0:
                ue0, ie0 = ue, ie
            else:
                uidx[e][s] = ue
                iidx[e][s] = ie
        uidx[0][s] = ue0
        iidx[0][s] = ie0
        return carry

    lax.fori_loop(0, KV, mkidx, 0)

    copies = []
    for e in range(EMB):
        copies.append(pltpu.async_copy(
            user_hbm.at[uidx[e]], ubuf[e], sem_u))
        copies.append(pltpu.async_copy(
            item_hbm.at[iidx[e]], ibuf[e], sem_i))
    for c in copies:
        c.wait()

    def chunk(k, carry):
        s = pl.ds(k * L, L)
        acc = ubuf[0][s] * ibuf[0][s]
        for e in range(1, EMB):
            acc = acc + ubuf[e][s] * ibuf[e][s]
        outv_v[s] = acc
        return carry

    lax.fori_loop(0, KV, chunk, 0)
    pltpu.sync_copy(outv_v, out_hbm.at[pl.ds(base, BPW)])


@jax.jit
def kernel(user_table, item_table, u_id, i_id):
    ut = user_table.T
    it = item_table.T
    ua = ut[:, RMAIN:].reshape(-1)
    ia = it[:, RMAIN:].reshape(-1)
    mesh = plsc.VectorSubcoreMesh(core_axis_name="c", subcore_axis_name="s",
                                  num_cores=NC, num_subcores=NS)

    detile = functools.partial(
        pl.kernel,
        out_type=[jax.ShapeDtypeStruct((FLATN,), jnp.float32),
                  jax.ShapeDtypeStruct((FLATN,), jnp.float32)],
        mesh=mesh,
        scratch_types=[
            pltpu.VMEM((8, SW), jnp.float32),
            pltpu.SemaphoreType.DMA,
            pltpu.SemaphoreType.DMA,
        ],
    )(_detile_body)
    uflat, iflat = detile(ut, it, ua, ia)

    gather = functools.partial(
        pl.kernel,
        out_type=jax.ShapeDtypeStruct((BATCH,), jnp.float32),
        mesh=mesh,
        scratch_types=(
            [pltpu.VMEM((BPW,), jnp.int32) for _ in range(2 * EMB)]
            + [pltpu.VMEM((BPW,), jnp.float32) for _ in range(2 * EMB)]
            + [pltpu.VMEM((BPW,), jnp.float32),
               pltpu.SemaphoreType.DMA,
               pltpu.SemaphoreType.DMA]
        ),
    )(_gather_body)
    return gather(uflat, iflat,
                  u_id.astype(jnp.int32), i_id.astype(jnp.int32))
